# Initial kernel scaffold; baseline (speedup 1.0000x reference)
import jax
import jax.numpy as jnp
from jax.experimental import pallas as pl


def kernel(x, edge_index, edge_weight, W0, b0, W1, b1):
    raise NotImplementedError("write your pallas kernel here")


# speedup vs baseline: 1.0131x; 1.0131x over previous
import functools

import jax
import jax.numpy as jnp
from jax.experimental import pallas as pl
from jax.experimental.pallas import tpu as pltpu

N = 10000
E = 320000
D = 128
ROW_BLK = 1000


def _proj_body(h_ref, f1_ref, f2_ref, wt_ref, b_ref, o_ref, *, act):
    z = (
        jnp.dot(h_ref[...], wt_ref[0:D], preferred_element_type=jnp.float32)
        + jnp.dot(f1_ref[...], wt_ref[D:2 * D], preferred_element_type=jnp.float32)
        + jnp.dot(f2_ref[...], wt_ref[2 * D:3 * D], preferred_element_type=jnp.float32)
        + b_ref[...]
    )
    if act:
        z = jnp.where(z >= 0, z, 0.01 * z)
    o_ref[...] = z


def _proj(h, f1, f2, W, b, act):
    wt = W.T
    out_d = W.shape[0]
    grid = (N // ROW_BLK,)
    rspec = pl.BlockSpec((ROW_BLK, D), lambda i: (i, 0))
    return pl.pallas_call(
        functools.partial(_proj_body, act=act),
        grid=grid,
        in_specs=[
            rspec, rspec, rspec,
            pl.BlockSpec((3 * D, out_d), lambda i: (0, 0)),
            pl.BlockSpec((out_d,), lambda i: (0,)),
        ],
        out_specs=pl.BlockSpec((ROW_BLK, out_d), lambda i: (i, 0)),
        out_shape=jax.ShapeDtypeStruct((N, out_d), jnp.float32),
    )(h, f1, f2, wt, b)


def _spmm(h, src, dst, w):
    return jax.ops.segment_sum(w[:, None] * h[src], dst, num_segments=N)


def kernel(x, edge_index, edge_weight, W0, b0, W1, b1):
    dst = edge_index[0]
    src = edge_index[1]
    f1 = _spmm(x, src, dst, edge_weight)
    f2 = _spmm(f1, src, dst, edge_weight)
    h = _proj(x, f1, f2, W0, b0, True)
    g1 = _spmm(h, src, dst, edge_weight)
    g2 = _spmm(g1, src, dst, edge_weight)
    return _proj(h, g1, g2, W1, b1, False)


# speedup vs baseline: 2.1136x; 2.0862x over previous
import functools

import jax
import jax.numpy as jnp
from jax import lax
from jax.experimental import pallas as pl
from jax.experimental.pallas import tpu as pltpu
from jax.experimental.pallas import tpu_sc as plsc

N = 10000
E = 320000
D = 128

NTILES = 32
NP = 10240
EPT = 10240
C = 128
NCH = EPT // C
RPT = NP // 16


def _sc_spmm(h, srcp, dstp, wp, zrows):
    mesh = plsc.VectorSubcoreMesh(core_axis_name="c", subcore_axis_name="s")

    @functools.partial(
        pl.kernel,
        out_type=jax.ShapeDtypeStruct((2, NP, D), jnp.float32),
        mesh=mesh,
        scratch_types=[
            pltpu.VMEM_SHARED((NP, D), jnp.float32),
            pltpu.VMEM((C,), jnp.int32),
            pltpu.VMEM((C,), jnp.int32),
            pltpu.VMEM((C,), jnp.float32),
            pltpu.VMEM((C, D), jnp.float32),
            pltpu.SemaphoreType.DMA,
        ],
    )
    def k(h_hbm, src_hbm, dst_hbm, w_hbm, z_hbm, out_hbm,
          acc, siv, div, wv, rows, sem):
        c = lax.axis_index("c")
        s = lax.axis_index("s")
        wid = s * 2 + c
        base = wid * EPT
        r0 = s * RPT

        pltpu.sync_copy(z_hbm, acc.at[pl.ds(r0, RPT)])
        plsc.subcore_barrier()

        def chunk(j, carry):
            off = base + j * C
            pltpu.sync_copy(src_hbm.at[pl.ds(off, C)], siv)
            pltpu.sync_copy(dst_hbm.at[pl.ds(off, C)], div)
            pltpu.sync_copy(w_hbm.at[pl.ds(off, C)], wv)
            pltpu.async_copy(h_hbm.at[siv], rows, sem).wait()

            def group(g, cc):
                wg = wv[pl.ds(g * 16, 16)]
                for i in range(16):
                    ww = wg[i]
                    e = g * 16 + i
                    for t in range(D // 16):
                        sl = pl.ds(t * 16, 16)
                        rows[e, sl] = rows[e, sl] * ww
                return cc

            lax.fori_loop(0, C // 16, group, 0)
            pltpu.sync_copy(rows, acc.at[div], add=True)
            return carry

        lax.fori_loop(0, NCH, chunk, 0)
        plsc.subcore_barrier()
        pltpu.sync_copy(acc.at[pl.ds(r0, RPT)], out_hbm.at[c, pl.ds(r0, RPT)])

    return k(h, srcp, dstp, wp, zrows)


def _combine_body(p_ref, o_ref):
    o_ref[...] = p_ref[0] + p_ref[1]


def _combine(p):
    blk = 2048
    return pl.pallas_call(
        _combine_body,
        grid=(NP // blk,),
        in_specs=[pl.BlockSpec((2, blk, D), lambda i: (0, i, 0))],
        out_specs=pl.BlockSpec((blk, D), lambda i: (i, 0)),
        out_shape=jax.ShapeDtypeStruct((NP, D), jnp.float32),
    )(p)


def _proj_body(h_ref, f1_ref, f2_ref, wt_ref, b_ref, o_ref, *, act):
    z = (
        jnp.dot(h_ref[...], wt_ref[0:D], preferred_element_type=jnp.float32)
        + jnp.dot(f1_ref[...], wt_ref[D:2 * D], preferred_element_type=jnp.float32)
        + jnp.dot(f2_ref[...], wt_ref[2 * D:3 * D], preferred_element_type=jnp.float32)
        + b_ref[...]
    )
    if act:
        z = jnp.where(z >= 0, z, 0.01 * z)
    o_ref[...] = z


def _proj(h, f1, f2, W, b, act, out_rows, blk):
    wt = W.T
    out_d = W.shape[0]
    rspec = pl.BlockSpec((blk, D), lambda i: (i, 0))
    return pl.pallas_call(
        functools.partial(_proj_body, act=act),
        grid=(out_rows // blk,),
        in_specs=[
            rspec, rspec, rspec,
            pl.BlockSpec((3 * D, out_d), lambda i: (0, 0)),
            pl.BlockSpec((out_d,), lambda i: (0,)),
        ],
        out_specs=pl.BlockSpec((blk, out_d), lambda i: (i, 0)),
        out_shape=jax.ShapeDtypeStruct((out_rows, out_d), jnp.float32),
    )(h, f1, f2, wt, b)


def kernel(x, edge_index, edge_weight, W0, b0, W1, b1):
    dst = edge_index[0]
    src = edge_index[1]
    pad = NTILES * EPT - E
    srcp = jnp.pad(src.astype(jnp.int32), (0, pad))
    dstp = jnp.pad(dst.astype(jnp.int32), (0, pad))
    wp = jnp.pad(edge_weight, (0, pad))
    zrows = jnp.zeros((RPT, D), jnp.float32)

    f1 = _combine(_sc_spmm(x, srcp, dstp, wp, zrows))
    f2 = _combine(_sc_spmm(f1, srcp, dstp, wp, zrows))
    h1 = _proj(x, f1, f2, W0, b0, True, NP, 2048)
    g1 = _combine(_sc_spmm(h1, srcp, dstp, wp, zrows))
    g2 = _combine(_sc_spmm(g1, srcp, dstp, wp, zrows))
    return _proj(h1, g1, g2, W1, b1, False, N, 1000)


# speedup vs baseline: 2.2824x; 1.0799x over previous
import functools

import jax
import jax.numpy as jnp
from jax import lax
from jax.experimental import pallas as pl
from jax.experimental.pallas import tpu as pltpu
from jax.experimental.pallas import tpu_sc as plsc

N = 10000
E = 320000
D = 128

NTILES = 32
NP = 10240
C = 96
NCH = 108
EPT = NCH * C
NB = NCH // 3
RPT = NP // 16


def _sc_spmm(h, srcp, dstp, wp, zrows):
    mesh = plsc.VectorSubcoreMesh(core_axis_name="c", subcore_axis_name="s")

    @functools.partial(
        pl.kernel,
        out_type=jax.ShapeDtypeStruct((2, NP, D), jnp.float32),
        mesh=mesh,
        scratch_types=[
            pltpu.VMEM_SHARED((NP, D), jnp.float32),
            pltpu.VMEM((3, 1, C), jnp.int32),
            pltpu.VMEM((3, 1, C), jnp.int32),
            pltpu.VMEM((3, 1, C), jnp.float32),
            pltpu.VMEM((3, C, D), jnp.float32),
            pltpu.SemaphoreType.DMA((3,)),
            pltpu.SemaphoreType.DMA((3,)),
            pltpu.SemaphoreType.DMA,
        ],
    )
    def k(h_hbm, src_hbm, dst2_hbm, w_hbm, z_hbm, out_hbm,
          acc, siv, div, wv, rows3, isems, gsems, ssem):
        c = lax.axis_index("c")
        s = lax.axis_index("s")
        wid = s * 2 + c
        base = wid * EPT
        r0 = s * RPT

        def eload(jj, b, wait):
            args = (
                (src_hbm.at[pl.ds(base + jj * C, C)], siv.at[b, 0], isems.at[b]),
                (dst2_hbm.at[wid * NCH + jj], div.at[b], isems.at[b]),
                (w_hbm.at[pl.ds(base + jj * C, C)], wv.at[b, 0], isems.at[b]),
            )
            for a in args:
                if wait:
                    pltpu.make_async_copy(*a).wait()
                else:
                    pltpu.async_copy(*a)

        def gather(jj, b, wait):
            if wait:
                pltpu.make_async_copy(
                    h_hbm.at[siv.at[b, 0]], rows3.at[b], gsems.at[b]).wait()
            else:
                pltpu.async_copy(
                    h_hbm.at[siv.at[b, 0]], rows3.at[b], gsems.at[b])

        def scat_wait(b):
            pltpu.make_async_copy(rows3.at[b], acc.at[div.at[b, 0]], ssem).wait()

        eload(0, 0, False)
        eload(1, 1, False)
        pltpu.sync_copy(z_hbm, acc.at[pl.ds(r0, RPT)])
        eload(0, 0, True)
        gather(0, 0, False)
        plsc.subcore_barrier()

        def outer(j, carry):
            for u in range(3):
                jj = 3 * j + u
                b0, b1, b2 = u, (u + 1) % 3, (u + 2) % 3

                @pl.when(jj + 1 < NCH)
                def _():
                    eload(jj + 1, b1, True)
                    gather(jj + 1, b1, False)

                @pl.when(jj >= 1)
                def _():
                    scat_wait(b2)

                @pl.when(jj + 2 < NCH)
                def _():
                    eload(jj + 2, b2, False)

                gather(jj, b0, True)

                def group(g, cc):
                    wg = wv[b0, 0, pl.ds(g * 16, 16)]
                    for i in range(16):
                        ww = wg[i]
                        e = g * 16 + i
                        for t in range(D // 16):
                            sl = pl.ds(t * 16, 16)
                            rows3[b0, e, sl] = rows3[b0, e, sl] * ww
                    return cc

                lax.fori_loop(0, C // 16, group, 0)
                pltpu.async_copy(rows3.at[b0], acc.at[div.at[b0, 0]], ssem, add=True)
            return carry

        lax.fori_loop(0, NB, outer, 0)
        scat_wait((NCH - 1) % 3)
        plsc.subcore_barrier()
        pltpu.sync_copy(acc.at[pl.ds(r0, RPT)], out_hbm.at[c, pl.ds(r0, RPT)])

    return k(h, srcp, dstp, wp, zrows)


def _combine_body(p_ref, o_ref):
    o_ref[...] = p_ref[0] + p_ref[1]


def _combine(p):
    blk = 2048
    return pl.pallas_call(
        _combine_body,
        grid=(NP // blk,),
        in_specs=[pl.BlockSpec((2, blk, D), lambda i: (0, i, 0))],
        out_specs=pl.BlockSpec((blk, D), lambda i: (i, 0)),
        out_shape=jax.ShapeDtypeStruct((NP, D), jnp.float32),
    )(p)


def _proj_body(h_ref, f1_ref, f2_ref, wt_ref, b_ref, o_ref, *, act):
    z = (
        jnp.dot(h_ref[...], wt_ref[0:D], preferred_element_type=jnp.float32)
        + jnp.dot(f1_ref[...], wt_ref[D:2 * D], preferred_element_type=jnp.float32)
        + jnp.dot(f2_ref[...], wt_ref[2 * D:3 * D], preferred_element_type=jnp.float32)
        + b_ref[...]
    )
    if act:
        z = jnp.where(z >= 0, z, 0.01 * z)
    o_ref[...] = z


def _proj(h, f1, f2, W, b, act, out_rows, blk):
    wt = W.T
    out_d = W.shape[0]
    rspec = pl.BlockSpec((blk, D), lambda i: (i, 0))
    return pl.pallas_call(
        functools.partial(_proj_body, act=act),
        grid=(out_rows // blk,),
        in_specs=[
            rspec, rspec, rspec,
            pl.BlockSpec((3 * D, out_d), lambda i: (0, 0)),
            pl.BlockSpec((out_d,), lambda i: (0,)),
        ],
        out_specs=pl.BlockSpec((blk, out_d), lambda i: (i, 0)),
        out_shape=jax.ShapeDtypeStruct((out_rows, out_d), jnp.float32),
    )(h, f1, f2, wt, b)


def kernel(x, edge_index, edge_weight, W0, b0, W1, b1):
    dst = edge_index[0]
    src = edge_index[1]
    pad = NTILES * EPT - E
    srcp = jnp.pad(src.astype(jnp.int32), (0, pad))
    dstp = jnp.pad(dst.astype(jnp.int32), (0, pad)).reshape(NTILES * NCH, 1, C)
    wp = jnp.pad(edge_weight, (0, pad))
    zrows = jnp.zeros((RPT, D), jnp.float32)

    f1 = _combine(_sc_spmm(x, srcp, dstp, wp, zrows))
    f2 = _combine(_sc_spmm(f1, srcp, dstp, wp, zrows))
    h1 = _proj(x, f1, f2, W0, b0, True, NP, 2048)
    g1 = _combine(_sc_spmm(h1, srcp, dstp, wp, zrows))
    g2 = _combine(_sc_spmm(g1, srcp, dstp, wp, zrows))
    return _proj(h1, g1, g2, W1, b1, False, N, 1000)
